# baseline (speedup 1.0000x reference)
import dataclasses
import functools

import jax
import jax.numpy as jnp
from jax import lax
from jax.experimental import pallas as pl
from jax.experimental.pallas import tpu as pltpu
from jax.experimental.pallas import tpu_sc as plsc

N_IN = 10000
N_OUT = 1000
B = 64
E_IN = 320000
E_OUT = 16000
D = 128
NUM_CLASSES = 10

NC = 2
NS = 16
NW = NC * NS

NPI = 10240
NPO = 1024
EPWI = E_IN // NW
KI = 80
EI_P = 327680
CHI = EI_P // NW // KI
EO_P = 16384
EPWO = EO_P // NW
KO = 64
CHO = EPWO // KO
RSI = NPI // NW
KSI = 80
CSI = RSI // KSI
RSO = NPO // NW
KSO = 32
CSO = 1

_mesh = plsc.VectorSubcoreMesh(core_axis_name="c", subcore_axis_name="s")
_sc_params = pltpu.CompilerParams()
if "needs_layout_passes" in pltpu.CompilerParams.__dataclass_fields__:
    _sc_params = dataclasses.replace(_sc_params, needs_layout_passes=False)
f32 = jnp.float32
i32 = jnp.int32


def _hist_body(src_i_hbm, dst_i_hbm, src_o_hbm, dst_o_hbm, val_o_hbm,
               ids_i_hbm, val_ii_hbm, ids_o_hbm, val_io_hbm,
               do_i_hbm, di_i_hbm, ci_hbm, do_o_hbm, di_o_hbm, co_hbm,
               ebuf_s, ebuf_d, obuf_s, obuf_d, obuf_v, ibuf, ivbuf,
               obuf2, ovbuf2,
               h_do_i, h_di_i, h_ci, h_do_o, h_di_o, h_co):
    w = lax.axis_index("c") * NS + lax.axis_index("s")
    ones = jnp.full((16,), 1.0, f32)
    zeros = jnp.zeros((16,), f32)

    @pl.loop(0, NPI // 16)
    def _(i):
        h_do_i[pl.ds(i * 16, 16)] = zeros
        h_di_i[pl.ds(i * 16, 16)] = zeros

    @pl.loop(0, NPO // 16)
    def _(i):
        h_ci[pl.ds(i * 16, 16)] = zeros
        h_do_o[pl.ds(i * 16, 16)] = zeros
        h_di_o[pl.ds(i * 16, 16)] = zeros

    @pl.loop(0, B // 16)
    def _(i):
        h_co[pl.ds(i * 16, 16)] = zeros

    pltpu.sync_copy(src_i_hbm.at[w], ebuf_s)
    pltpu.sync_copy(dst_i_hbm.at[w], ebuf_d)

    @pl.loop(0, EPWI // 16)
    def _(i):
        s16 = ebuf_s[pl.ds(i * 16, 16)]
        d16 = ebuf_d[pl.ds(i * 16, 16)]
        plsc.addupdate_scatter(h_do_i, [s16], ones)
        plsc.addupdate_scatter(h_di_i, [d16], ones)

    pltpu.sync_copy(src_o_hbm.at[w], obuf_s)
    pltpu.sync_copy(dst_o_hbm.at[w], obuf_d)
    pltpu.sync_copy(val_o_hbm.at[w], obuf_v)

    @pl.loop(0, EPWO // 16)
    def _(i):
        s16 = obuf_s[pl.ds(i * 16, 16)]
        d16 = obuf_d[pl.ds(i * 16, 16)]
        v16 = obuf_v[pl.ds(i * 16, 16)]
        plsc.addupdate_scatter(h_do_o, [s16], v16)
        plsc.addupdate_scatter(h_di_o, [d16], v16)

    pltpu.sync_copy(ids_i_hbm.at[w], ibuf)
    pltpu.sync_copy(val_ii_hbm.at[w], ivbuf)

    @pl.loop(0, RSI // 16)
    def _(i):
        g16 = ibuf[pl.ds(i * 16, 16)]
        v16 = ivbuf[pl.ds(i * 16, 16)]
        plsc.addupdate_scatter(h_ci, [g16], v16)

    pltpu.sync_copy(ids_o_hbm.at[w], obuf2)
    pltpu.sync_copy(val_io_hbm.at[w], ovbuf2)

    @pl.loop(0, RSO // 16)
    def _(i):
        g16 = obuf2[pl.ds(i * 16, 16)]
        v16 = ovbuf2[pl.ds(i * 16, 16)]
        plsc.addupdate_scatter(h_co, [g16], v16)

    pltpu.sync_copy(h_do_i, do_i_hbm.at[w])
    pltpu.sync_copy(h_di_i, di_i_hbm.at[w])
    pltpu.sync_copy(h_ci, ci_hbm.at[w])
    pltpu.sync_copy(h_do_o, do_o_hbm.at[w])
    pltpu.sync_copy(h_di_o, di_o_hbm.at[w])
    pltpu.sync_copy(h_co, co_hbm.at[w])


_hist_kernel = pl.kernel(
    _hist_body,
    out_type=(
        jax.ShapeDtypeStruct((NW, NPI), f32),
        jax.ShapeDtypeStruct((NW, NPI), f32),
        jax.ShapeDtypeStruct((NW, NPO), f32),
        jax.ShapeDtypeStruct((NW, NPO), f32),
        jax.ShapeDtypeStruct((NW, NPO), f32),
        jax.ShapeDtypeStruct((NW, B), f32),
    ),
    mesh=_mesh,
    compiler_params=_sc_params,
    scratch_types=[
        pltpu.VMEM((EPWI,), i32), pltpu.VMEM((EPWI,), i32),
        pltpu.VMEM((EPWO,), i32), pltpu.VMEM((EPWO,), i32),
        pltpu.VMEM((EPWO,), f32),
        pltpu.VMEM((RSI,), i32), pltpu.VMEM((RSI,), f32),
        pltpu.VMEM((RSO,), i32), pltpu.VMEM((RSO,), f32),
        pltpu.VMEM((NPI,), f32), pltpu.VMEM((NPI,), f32),
        pltpu.VMEM((NPO,), f32), pltpu.VMEM((NPO,), f32),
        pltpu.VMEM((NPO,), f32), pltpu.VMEM((B,), f32),
    ],
)


def _make_mp(np_, ch, k):
    rz = np_ // NS

    def body(h_hbm, pk_hbm, zeros_hbm, out_hbm, pk_v, rows_v, acc_sh):
        c_id = lax.axis_index("c")
        s_id = lax.axis_index("s")
        w = c_id * NS + s_id
        pltpu.sync_copy(zeros_hbm.at[pl.ds(s_id * rz, rz)],
                        acc_sh.at[pl.ds(s_id * rz, rz)])
        plsc.subcore_barrier()

        @pl.loop(0, ch)
        def _(c):
            pltpu.sync_copy(pk_hbm.at[w].at[c], pk_v)
            pltpu.sync_copy(h_hbm.at[pk_v.at[0]], rows_v)

            @pl.loop(0, k // 16)
            def _(ii):
                wv = plsc.bitcast(pk_v[2, pl.ds(ii * 16, 16)], f32)
                for r in range(16):
                    wgt = wv[r]
                    for j in range(D // 16):
                        sl = (ii * 16 + r, pl.ds(j * 16, 16))
                        rows_v[sl] = rows_v[sl] * wgt

            pltpu.sync_copy(rows_v, acc_sh.at[pk_v.at[1]], add=True)

        plsc.subcore_barrier()
        pltpu.sync_copy(acc_sh.at[pl.ds(s_id * rz, rz)],
                        out_hbm.at[c_id].at[pl.ds(s_id * rz, rz)])

    return pl.kernel(
        body,
        out_type=jax.ShapeDtypeStruct((NC, np_, D), f32),
        mesh=_mesh,
        compiler_params=_sc_params,
        scratch_types=[
            pltpu.VMEM((3, k), i32), pltpu.VMEM((k, D), f32),
            pltpu.VMEM_SHARED((np_, D), f32),
        ],
    )


_mp_inner = _make_mp(NPI, CHI, KI)
_mp_outer = _make_mp(NPO, CHO, KO)


def _make_segsum(np_rows, m, cs, k):
    rpw = np_rows // NW
    rz = m // NS

    def body(h_hbm, ids_hbm, zeros_hbm, out_hbm, ids_v, rows_v, acc_sh):
        c_id = lax.axis_index("c")
        s_id = lax.axis_index("s")
        w = c_id * NS + s_id
        pltpu.sync_copy(zeros_hbm.at[pl.ds(s_id * rz, rz)],
                        acc_sh.at[pl.ds(s_id * rz, rz)])
        pltpu.sync_copy(ids_hbm.at[w], ids_v)
        plsc.subcore_barrier()

        @pl.loop(0, cs)
        def _(c):
            pltpu.sync_copy(h_hbm.at[pl.ds(w * rpw + c * k, k)], rows_v)
            pltpu.sync_copy(rows_v, acc_sh.at[ids_v.at[c]], add=True)

        plsc.subcore_barrier()
        pltpu.sync_copy(acc_sh.at[pl.ds(s_id * rz, rz)],
                        out_hbm.at[c_id].at[pl.ds(s_id * rz, rz)])

    return pl.kernel(
        body,
        out_type=jax.ShapeDtypeStruct((NC, m, D), f32),
        mesh=_mesh,
        compiler_params=_sc_params,
        scratch_types=[
            pltpu.VMEM((cs, k), i32), pltpu.VMEM((k, D), f32),
            pltpu.VMEM_SHARED((m, D), f32),
        ],
    )


_segsum_inner = _make_segsum(NPI, NPO, CSI, KSI)
_segsum_outer = _make_segsum(NPO, B, CSO, KSO)


def _t0_body(do_i, di_i, ci, do_o, di_o, co, x, w1,
             h1_o, nsi_o, ndi_o, ici_o, nso_o, ndo_o, ico_o):
    deg_out_i = jnp.sum(do_i[...], axis=0)
    deg_in_i = jnp.sum(di_i[...], axis=0)
    cnt_i = jnp.sum(ci[...], axis=0)
    deg_out_o = jnp.sum(do_o[...], axis=0)
    deg_in_o = jnp.sum(di_o[...], axis=0)
    cnt_o = jnp.sum(co[...], axis=0)
    nsi = lax.rsqrt(jnp.maximum(deg_out_i, 1.0))
    ndi = lax.rsqrt(jnp.maximum(deg_in_i, 1.0))
    nso = lax.rsqrt(jnp.maximum(deg_out_o, 1.0))
    ndo = lax.rsqrt(jnp.maximum(deg_in_o, 1.0))
    nsi_o[...] = nsi
    ndi_o[...] = ndi
    ici_o[...] = 1.0 / jnp.maximum(cnt_i, 1.0)
    nso_o[...] = nso
    ndo_o[...] = ndo
    ico_o[...] = 1.0 / jnp.maximum(cnt_o, 1.0)
    h1_o[...] = jnp.dot(x[...] * nsi[:, None], w1[...],
                        preferred_element_type=f32)


def _tc_t0(do_i, di_i, ci, do_o, di_o, co, x, w1):
    return pl.pallas_call(
        _t0_body,
        out_shape=(
            jax.ShapeDtypeStruct((NPI, D), f32),
            jax.ShapeDtypeStruct((NPI,), f32),
            jax.ShapeDtypeStruct((NPI,), f32),
            jax.ShapeDtypeStruct((NPO,), f32),
            jax.ShapeDtypeStruct((NPO,), f32),
            jax.ShapeDtypeStruct((NPO,), f32),
            jax.ShapeDtypeStruct((B,), f32),
        ),
    )(do_i, di_i, ci, do_o, di_o, co, x, w1)


def _tmid_body(p, nd, ns, b, w, o):
    agg = p[0] + p[1]
    x = jax.nn.relu(agg * nd[...][:, None] + b[...][None, :]) * ns[...][:, None]
    o[...] = jnp.dot(x, w[...], preferred_element_type=f32)


def _tc_mid(p, nd, ns, b, w):
    n = p.shape[1]
    return pl.pallas_call(
        _tmid_body,
        out_shape=jax.ShapeDtypeStruct((n, w.shape[1]), f32),
    )(p, nd, ns, b, w)


def _tmask_body(p, nd, b, nreal_ref, o):
    agg = p[0] + p[1]
    h = jax.nn.relu(agg * nd[...][:, None] + b[...][None, :])
    rows = lax.broadcasted_iota(i32, h.shape, 0)
    o[...] = jnp.where(rows < nreal_ref[0], h, 0.0)


def _tc_mask(p, nd, b, nreal):
    n = p.shape[1]
    return pl.pallas_call(
        _tmask_body,
        out_shape=jax.ShapeDtypeStruct((n, D), f32),
    )(p, nd, b, jnp.full((1,), nreal, i32))


def _t3_body(q, ici, feat, nso, w3, o):
    red = (q[0] + q[1]) * ici[...][:, None]
    merged = jnp.concatenate([feat[...], red], axis=1)
    o[...] = jnp.dot(merged * nso[...][:, None], w3[...],
                     preferred_element_type=f32)


def _tc_t3(q, ici, feat, nso, w3):
    return pl.pallas_call(
        _t3_body,
        out_shape=jax.ShapeDtypeStruct((NPO, D), f32),
    )(q, ici, feat, nso, w3)


def _t8_body(s, ico, wc, bc, o):
    hg = (s[0] + s[1]) * ico[...][:, None]
    o[...] = jnp.dot(hg, wc[...], preferred_element_type=f32) + bc[...]


def _tc_t8(s, ico, wc, bc):
    return pl.pallas_call(
        _t8_body,
        out_shape=jax.ShapeDtypeStruct((B, NUM_CLASSES), f32),
    )(s, ico, wc, bc[None, :])


def kernel(in_layer_feat, out_layer_feat, edge_index_in, edge_index_out,
           node_graph_ids_in, graph_ids_out, inner_edge_weight,
           outer_edge_weight, W1, b1, W2, b2, W3, b3, W4, b4, W5, b5,
           W6, b6, Wc, bc):
    src_i = edge_index_in[0].astype(i32)
    dst_i = edge_index_in[1].astype(i32)
    src_o = edge_index_out[0].astype(i32)
    dst_o = edge_index_out[1].astype(i32)

    x_p = jnp.pad(in_layer_feat, ((0, NPI - N_IN), (0, 0)))
    feat_o_p = jnp.pad(out_layer_feat, ((0, NPO - N_OUT), (0, 0)))
    zeros_big = jnp.zeros((NPI, D), f32)

    pad_ei = EI_P - E_IN
    src_ip = jnp.pad(src_i, (0, pad_ei))
    dst_ip = jnp.pad(dst_i, (0, pad_ei))
    ew_ip = jnp.pad(inner_edge_weight.astype(f32), (0, pad_ei))
    ew_i_bits = lax.bitcast_convert_type(ew_ip, i32)
    pk_i = jnp.stack([src_ip.reshape(NW, CHI, KI),
                      dst_ip.reshape(NW, CHI, KI),
                      ew_i_bits.reshape(NW, CHI, KI)], axis=2)
    srcs_i2 = src_i.reshape(NW, EPWI)
    dsts_i2 = dst_i.reshape(NW, EPWI)

    pad_e = EO_P - E_OUT
    src_o_p = jnp.pad(src_o, (0, pad_e))
    dst_o_p = jnp.pad(dst_o, (0, pad_e))
    ew_o_p = jnp.pad(outer_edge_weight.astype(f32), (0, pad_e))
    val_o_p = jnp.pad(jnp.ones((E_OUT,), f32), (0, pad_e))
    ew_o_bits = lax.bitcast_convert_type(ew_o_p, i32)
    pk_o = jnp.stack([src_o_p.reshape(NW, CHO, KO),
                      dst_o_p.reshape(NW, CHO, KO),
                      ew_o_bits.reshape(NW, CHO, KO)], axis=2)
    srcs_o2 = src_o_p.reshape(NW, EPWO)
    dsts_o2 = dst_o_p.reshape(NW, EPWO)
    vals_o2 = val_o_p.reshape(NW, EPWO)

    ids_i_p = jnp.pad(node_graph_ids_in.astype(i32), (0, NPI - N_IN))
    val_ii = jnp.pad(jnp.ones((N_IN,), f32), (0, NPI - N_IN))
    ids_o_p = jnp.pad(graph_ids_out.astype(i32), (0, NPO - N_OUT))
    val_io = jnp.pad(jnp.ones((N_OUT,), f32), (0, NPO - N_OUT))
    ids_i2 = ids_i_p.reshape(NW, RSI)
    val_ii2 = val_ii.reshape(NW, RSI)
    ids_o2 = ids_o_p.reshape(NW, RSO)
    val_io2 = val_io.reshape(NW, RSO)
    ids_i3 = ids_i_p.reshape(NW, CSI, KSI)
    ids_o3 = ids_o_p.reshape(NW, CSO, KSO)

    do_i, di_i, ci, do_o, di_o, co = _hist_kernel(
        srcs_i2, dsts_i2, srcs_o2, dsts_o2, vals_o2,
        ids_i2, val_ii2, ids_o2, val_io2)

    h1, nsi, ndi, ici, nso, ndo, ico = _tc_t0(
        do_i, di_i, ci, do_o, di_o, co, x_p, W1)

    p1 = _mp_inner(h1, pk_i, zeros_big)
    h2 = _tc_mid(p1, ndi, nsi, b1, W2)
    p2 = _mp_inner(h2, pk_i, zeros_big)
    hin = _tc_mask(p2, ndi, b2, N_IN)

    q = _segsum_inner(hin, ids_i3, zeros_big)
    h3 = _tc_t3(q, ici, feat_o_p, nso, W3)

    p3 = _mp_outer(h3, pk_o, zeros_big)
    h4 = _tc_mid(p3, ndo, nso, b3, W4)
    p4 = _mp_outer(h4, pk_o, zeros_big)
    h5 = _tc_mid(p4, ndo, nso, b4, W5)
    p5 = _mp_outer(h5, pk_o, zeros_big)
    h6 = _tc_mid(p5, ndo, nso, b5, W6)
    p6 = _mp_outer(h6, pk_o, zeros_big)
    hout = _tc_mask(p6, ndo, b6, N_OUT)

    s = _segsum_outer(hout, ids_o3, zeros_big)
    return _tc_t8(s, ico, Wc, bc)


# speedup vs baseline: 2.2534x; 2.2534x over previous
import dataclasses
import functools

import jax
import jax.numpy as jnp
from jax import lax
from jax.experimental import pallas as pl
from jax.experimental.pallas import tpu as pltpu
from jax.experimental.pallas import tpu_sc as plsc

N_IN = 10000
N_OUT = 1000
B = 64
E_IN = 320000
E_OUT = 16000
D = 128
NUM_CLASSES = 10

NC = 2
NS = 16
NW = NC * NS

NPI = 10240
NPO = 1024
EPWI = E_IN // NW
KI = 128
EI_P = 327680
CHI = EI_P // NW // KI
EO_P = 16384
EPWO = EO_P // NW
KO = 128
CHO = EPWO // KO
RSI = NPI // NW
KSI = 80
CSI = RSI // KSI
RSO = NPO // NW
KSO = 32
CSO = 1

_mesh = plsc.VectorSubcoreMesh(core_axis_name="c", subcore_axis_name="s")
_sc_params = pltpu.CompilerParams()
if "needs_layout_passes" in pltpu.CompilerParams.__dataclass_fields__:
    _sc_params = dataclasses.replace(_sc_params, needs_layout_passes=False)
f32 = jnp.float32
i32 = jnp.int32


def _hist_body(src_i_hbm, dst_i_hbm, src_o_hbm, dst_o_hbm, val_o_hbm,
               ids_i_hbm, val_ii_hbm, ids_o_hbm, val_io_hbm,
               do_i_hbm, di_i_hbm, ci_hbm, do_o_hbm, di_o_hbm, co_hbm,
               ebuf_s, ebuf_d, obuf_s, obuf_d, obuf_v, ibuf, ivbuf,
               obuf2, ovbuf2,
               h_do_i, h_di_i, h_ci, h_do_o, h_di_o, h_co):
    w = lax.axis_index("c") * NS + lax.axis_index("s")
    ones = jnp.full((16,), 1.0, f32)
    zeros = jnp.zeros((16,), f32)

    @pl.loop(0, NPI // 16)
    def _(i):
        h_do_i[pl.ds(i * 16, 16)] = zeros
        h_di_i[pl.ds(i * 16, 16)] = zeros

    @pl.loop(0, NPO // 16)
    def _(i):
        h_ci[pl.ds(i * 16, 16)] = zeros
        h_do_o[pl.ds(i * 16, 16)] = zeros
        h_di_o[pl.ds(i * 16, 16)] = zeros

    @pl.loop(0, B // 16)
    def _(i):
        h_co[pl.ds(i * 16, 16)] = zeros

    pltpu.sync_copy(src_i_hbm.at[w], ebuf_s)
    pltpu.sync_copy(dst_i_hbm.at[w], ebuf_d)

    @pl.loop(0, EPWI // 16)
    def _(i):
        s16 = ebuf_s[pl.ds(i * 16, 16)]
        d16 = ebuf_d[pl.ds(i * 16, 16)]
        plsc.addupdate_scatter(h_do_i, [s16], ones)
        plsc.addupdate_scatter(h_di_i, [d16], ones)

    pltpu.sync_copy(src_o_hbm.at[w], obuf_s)
    pltpu.sync_copy(dst_o_hbm.at[w], obuf_d)
    pltpu.sync_copy(val_o_hbm.at[w], obuf_v)

    @pl.loop(0, EPWO // 16)
    def _(i):
        s16 = obuf_s[pl.ds(i * 16, 16)]
        d16 = obuf_d[pl.ds(i * 16, 16)]
        v16 = obuf_v[pl.ds(i * 16, 16)]
        plsc.addupdate_scatter(h_do_o, [s16], v16)
        plsc.addupdate_scatter(h_di_o, [d16], v16)

    pltpu.sync_copy(ids_i_hbm.at[w], ibuf)
    pltpu.sync_copy(val_ii_hbm.at[w], ivbuf)

    @pl.loop(0, RSI // 16)
    def _(i):
        g16 = ibuf[pl.ds(i * 16, 16)]
        v16 = ivbuf[pl.ds(i * 16, 16)]
        plsc.addupdate_scatter(h_ci, [g16], v16)

    pltpu.sync_copy(ids_o_hbm.at[w], obuf2)
    pltpu.sync_copy(val_io_hbm.at[w], ovbuf2)

    @pl.loop(0, RSO // 16)
    def _(i):
        g16 = obuf2[pl.ds(i * 16, 16)]
        v16 = ovbuf2[pl.ds(i * 16, 16)]
        plsc.addupdate_scatter(h_co, [g16], v16)

    pltpu.sync_copy(h_do_i, do_i_hbm.at[w])
    pltpu.sync_copy(h_di_i, di_i_hbm.at[w])
    pltpu.sync_copy(h_ci, ci_hbm.at[w])
    pltpu.sync_copy(h_do_o, do_o_hbm.at[w])
    pltpu.sync_copy(h_di_o, di_o_hbm.at[w])
    pltpu.sync_copy(h_co, co_hbm.at[w])


_hist_kernel = pl.kernel(
    _hist_body,
    out_type=(
        jax.ShapeDtypeStruct((NW, NPI), f32),
        jax.ShapeDtypeStruct((NW, NPI), f32),
        jax.ShapeDtypeStruct((NW, NPO), f32),
        jax.ShapeDtypeStruct((NW, NPO), f32),
        jax.ShapeDtypeStruct((NW, NPO), f32),
        jax.ShapeDtypeStruct((NW, B), f32),
    ),
    mesh=_mesh,
    compiler_params=_sc_params,
    scratch_types=[
        pltpu.VMEM((EPWI,), i32), pltpu.VMEM((EPWI,), i32),
        pltpu.VMEM((EPWO,), i32), pltpu.VMEM((EPWO,), i32),
        pltpu.VMEM((EPWO,), f32),
        pltpu.VMEM((RSI,), i32), pltpu.VMEM((RSI,), f32),
        pltpu.VMEM((RSO,), i32), pltpu.VMEM((RSO,), f32),
        pltpu.VMEM((NPI,), f32), pltpu.VMEM((NPI,), f32),
        pltpu.VMEM((NPO,), f32), pltpu.VMEM((NPO,), f32),
        pltpu.VMEM((NPO,), f32), pltpu.VMEM((B,), f32),
    ],
)


def _make_mp(np_, ch, k):
    rz = np_ // NS

    def body(h_hbm, pk_hbm, zeros_hbm, out_hbm, pk_v, rows_v, acc_sh):
        c_id = lax.axis_index("c")
        s_id = lax.axis_index("s")
        w = c_id * NS + s_id
        pltpu.sync_copy(zeros_hbm.at[pl.ds(s_id * rz, rz)],
                        acc_sh.at[pl.ds(s_id * rz, rz)])
        plsc.subcore_barrier()

        @pl.loop(0, ch)
        def _(c):
            pltpu.sync_copy(pk_hbm.at[w].at[c], pk_v)
            pltpu.sync_copy(h_hbm.at[pk_v.at[0]], rows_v)

            @pl.loop(0, k // 16)
            def _(ii):
                wv = plsc.bitcast(pk_v[2, pl.ds(ii * 16, 16)], f32)
                for r in range(16):
                    wgt = wv[r]
                    for j in range(D // 16):
                        sl = (ii * 16 + r, pl.ds(j * 16, 16))
                        rows_v[sl] = rows_v[sl] * wgt

            pltpu.sync_copy(rows_v, acc_sh.at[pk_v.at[1]], add=True)

        plsc.subcore_barrier()
        pltpu.sync_copy(acc_sh.at[pl.ds(s_id * rz, rz)],
                        out_hbm.at[c_id].at[pl.ds(s_id * rz, rz)])

    return pl.kernel(
        body,
        out_type=jax.ShapeDtypeStruct((NC, np_, D), f32),
        mesh=_mesh,
        compiler_params=_sc_params,
        scratch_types=[
            pltpu.VMEM((3, k), i32), pltpu.VMEM((k, D), f32),
            pltpu.VMEM_SHARED((np_, D), f32),
        ],
    )


_mp_inner = _make_mp(NPI, CHI, KI)
_mp_outer = _make_mp(NPO, CHO, KO)


def _make_segsum(np_rows, m, cs, k):
    rpw = np_rows // NW
    rz = m // NS

    def body(h_hbm, ids_hbm, zeros_hbm, out_hbm, ids_v, rows_v, acc_sh):
        c_id = lax.axis_index("c")
        s_id = lax.axis_index("s")
        w = c_id * NS + s_id
        pltpu.sync_copy(zeros_hbm.at[pl.ds(s_id * rz, rz)],
                        acc_sh.at[pl.ds(s_id * rz, rz)])
        pltpu.sync_copy(ids_hbm.at[w], ids_v)
        plsc.subcore_barrier()

        @pl.loop(0, cs)
        def _(c):
            pltpu.sync_copy(h_hbm.at[pl.ds(w * rpw + c * k, k)], rows_v)
            pltpu.sync_copy(rows_v, acc_sh.at[ids_v.at[c]], add=True)

        plsc.subcore_barrier()
        pltpu.sync_copy(acc_sh.at[pl.ds(s_id * rz, rz)],
                        out_hbm.at[c_id].at[pl.ds(s_id * rz, rz)])

    return pl.kernel(
        body,
        out_type=jax.ShapeDtypeStruct((NC, m, D), f32),
        mesh=_mesh,
        compiler_params=_sc_params,
        scratch_types=[
            pltpu.VMEM((cs, k), i32), pltpu.VMEM((k, D), f32),
            pltpu.VMEM_SHARED((m, D), f32),
        ],
    )


_segsum_inner = _make_segsum(NPI, NPO, CSI, KSI)
_segsum_outer = _make_segsum(NPO, B, CSO, KSO)


def _t0_body(do_i, di_i, ci, do_o, di_o, co, x, w1,
             h1_o, nsi_o, ndi_o, ici_o, nso_o, ndo_o, ico_o):
    deg_out_i = jnp.sum(do_i[...], axis=0)
    deg_in_i = jnp.sum(di_i[...], axis=0)
    cnt_i = jnp.sum(ci[...], axis=0)
    deg_out_o = jnp.sum(do_o[...], axis=0)
    deg_in_o = jnp.sum(di_o[...], axis=0)
    cnt_o = jnp.sum(co[...], axis=0)
    nsi = lax.rsqrt(jnp.maximum(deg_out_i, 1.0))
    ndi = lax.rsqrt(jnp.maximum(deg_in_i, 1.0))
    nso = lax.rsqrt(jnp.maximum(deg_out_o, 1.0))
    ndo = lax.rsqrt(jnp.maximum(deg_in_o, 1.0))
    nsi_o[...] = nsi
    ndi_o[...] = ndi
    ici_o[...] = 1.0 / jnp.maximum(cnt_i, 1.0)
    nso_o[...] = nso
    ndo_o[...] = ndo
    ico_o[...] = 1.0 / jnp.maximum(cnt_o, 1.0)
    h1_o[...] = jnp.dot(x[...] * nsi[:, None], w1[...],
                        preferred_element_type=f32)


def _tc_t0(do_i, di_i, ci, do_o, di_o, co, x, w1):
    return pl.pallas_call(
        _t0_body,
        out_shape=(
            jax.ShapeDtypeStruct((NPI, D), f32),
            jax.ShapeDtypeStruct((NPI,), f32),
            jax.ShapeDtypeStruct((NPI,), f32),
            jax.ShapeDtypeStruct((NPO,), f32),
            jax.ShapeDtypeStruct((NPO,), f32),
            jax.ShapeDtypeStruct((NPO,), f32),
            jax.ShapeDtypeStruct((B,), f32),
        ),
    )(do_i, di_i, ci, do_o, di_o, co, x, w1)


def _tmid_body(p, nd, ns, b, w, o):
    agg = p[0] + p[1]
    x = jax.nn.relu(agg * nd[...][:, None] + b[...][None, :]) * ns[...][:, None]
    o[...] = jnp.dot(x, w[...], preferred_element_type=f32)


def _tc_mid(p, nd, ns, b, w):
    n = p.shape[1]
    return pl.pallas_call(
        _tmid_body,
        out_shape=jax.ShapeDtypeStruct((n, w.shape[1]), f32),
    )(p, nd, ns, b, w)


def _tmask_body(p, nd, b, nreal_ref, o):
    agg = p[0] + p[1]
    h = jax.nn.relu(agg * nd[...][:, None] + b[...][None, :])
    rows = lax.broadcasted_iota(i32, h.shape, 0)
    o[...] = jnp.where(rows < nreal_ref[0], h, 0.0)


def _tc_mask(p, nd, b, nreal):
    n = p.shape[1]
    return pl.pallas_call(
        _tmask_body,
        out_shape=jax.ShapeDtypeStruct((n, D), f32),
    )(p, nd, b, jnp.full((1,), nreal, i32))


def _t3_body(q, ici, feat, nso, w3, o):
    red = (q[0] + q[1]) * ici[...][:, None]
    merged = jnp.concatenate([feat[...], red], axis=1)
    o[...] = jnp.dot(merged * nso[...][:, None], w3[...],
                     preferred_element_type=f32)


def _tc_t3(q, ici, feat, nso, w3):
    return pl.pallas_call(
        _t3_body,
        out_shape=jax.ShapeDtypeStruct((NPO, D), f32),
    )(q, ici, feat, nso, w3)


def _t8_body(s, ico, wc, bc, o):
    hg = (s[0] + s[1]) * ico[...][:, None]
    o[...] = jnp.dot(hg, wc[...], preferred_element_type=f32) + bc[...]


def _tc_t8(s, ico, wc, bc):
    return pl.pallas_call(
        _t8_body,
        out_shape=jax.ShapeDtypeStruct((B, NUM_CLASSES), f32),
    )(s, ico, wc, bc[None, :])


def kernel(in_layer_feat, out_layer_feat, edge_index_in, edge_index_out,
           node_graph_ids_in, graph_ids_out, inner_edge_weight,
           outer_edge_weight, W1, b1, W2, b2, W3, b3, W4, b4, W5, b5,
           W6, b6, Wc, bc):
    src_i = edge_index_in[0].astype(i32)
    dst_i = edge_index_in[1].astype(i32)
    src_o = edge_index_out[0].astype(i32)
    dst_o = edge_index_out[1].astype(i32)

    x_p = jnp.pad(in_layer_feat, ((0, NPI - N_IN), (0, 0)))
    feat_o_p = jnp.pad(out_layer_feat, ((0, NPO - N_OUT), (0, 0)))
    zeros_big = jnp.zeros((NPI, D), f32)

    pad_ei = EI_P - E_IN
    spread_i = N_IN + (jnp.arange(pad_ei, dtype=i32) % (NPI - N_IN))
    src_ip = jnp.concatenate([src_i, spread_i])
    dst_ip = jnp.concatenate([dst_i, spread_i])
    ew_ip = jnp.pad(inner_edge_weight.astype(f32), (0, pad_ei))
    ew_i_bits = lax.bitcast_convert_type(ew_ip, i32)
    pk_i = jnp.stack([src_ip.reshape(NW, CHI, KI),
                      dst_ip.reshape(NW, CHI, KI),
                      ew_i_bits.reshape(NW, CHI, KI)], axis=2)
    srcs_i2 = src_i.reshape(NW, EPWI)
    dsts_i2 = dst_i.reshape(NW, EPWI)

    pad_e = EO_P - E_OUT
    spread_o = N_OUT + (jnp.arange(pad_e, dtype=i32) % (NPO - N_OUT))
    src_o_p = jnp.concatenate([src_o, spread_o])
    dst_o_p = jnp.concatenate([dst_o, spread_o])
    ew_o_p = jnp.pad(outer_edge_weight.astype(f32), (0, pad_e))
    val_o_p = jnp.pad(jnp.ones((E_OUT,), f32), (0, pad_e))
    ew_o_bits = lax.bitcast_convert_type(ew_o_p, i32)
    pk_o = jnp.stack([src_o_p.reshape(NW, CHO, KO),
                      dst_o_p.reshape(NW, CHO, KO),
                      ew_o_bits.reshape(NW, CHO, KO)], axis=2)
    srcs_o2 = src_o_p.reshape(NW, EPWO)
    dsts_o2 = dst_o_p.reshape(NW, EPWO)
    vals_o2 = val_o_p.reshape(NW, EPWO)

    ids_i_p = jnp.concatenate([
        node_graph_ids_in.astype(i32),
        jnp.arange(NPI - N_IN, dtype=i32) % NPO])
    val_ii = jnp.pad(jnp.ones((N_IN,), f32), (0, NPI - N_IN))
    ids_o_p = jnp.concatenate([
        graph_ids_out.astype(i32),
        jnp.arange(NPO - N_OUT, dtype=i32) % B])
    val_io = jnp.pad(jnp.ones((N_OUT,), f32), (0, NPO - N_OUT))
    ids_i2 = ids_i_p.reshape(NW, RSI)
    val_ii2 = val_ii.reshape(NW, RSI)
    ids_o2 = ids_o_p.reshape(NW, RSO)
    val_io2 = val_io.reshape(NW, RSO)
    ids_i3 = ids_i_p.reshape(NW, CSI, KSI)
    ids_o3 = ids_o_p.reshape(NW, CSO, KSO)

    do_i, di_i, ci, do_o, di_o, co = _hist_kernel(
        srcs_i2, dsts_i2, srcs_o2, dsts_o2, vals_o2,
        ids_i2, val_ii2, ids_o2, val_io2)

    h1, nsi, ndi, ici, nso, ndo, ico = _tc_t0(
        do_i, di_i, ci, do_o, di_o, co, x_p, W1)

    p1 = _mp_inner(h1, pk_i, zeros_big)
    h2 = _tc_mid(p1, ndi, nsi, b1, W2)
    p2 = _mp_inner(h2, pk_i, zeros_big)
    hin = _tc_mask(p2, ndi, b2, N_IN)

    q = _segsum_inner(hin, ids_i3, zeros_big)
    h3 = _tc_t3(q, ici, feat_o_p, nso, W3)

    p3 = _mp_outer(h3, pk_o, zeros_big)
    h4 = _tc_mid(p3, ndo, nso, b3, W4)
    p4 = _mp_outer(h4, pk_o, zeros_big)
    h5 = _tc_mid(p4, ndo, nso, b4, W5)
    p5 = _mp_outer(h5, pk_o, zeros_big)
    h6 = _tc_mid(p5, ndo, nso, b5, W6)
    p6 = _mp_outer(h6, pk_o, zeros_big)
    hout = _tc_mask(p6, ndo, b6, N_OUT)

    s = _segsum_outer(hout, ids_o3, zeros_big)
    return _tc_t8(s, ico, Wc, bc)


# speedup vs baseline: 3.5702x; 1.5844x over previous
import dataclasses
import functools

import jax
import jax.numpy as jnp
from jax import lax
from jax.experimental import pallas as pl
from jax.experimental.pallas import tpu as pltpu
from jax.experimental.pallas import tpu_sc as plsc

N_IN = 10000
N_OUT = 1000
B = 64
E_IN = 320000
E_OUT = 16000
D = 128
NUM_CLASSES = 10

NC = 2
NS = 16
NW = NC * NS

NPI = 10240
NPO = 1024
EPWI = E_IN // NW
KI = 128
EI_P = 327680
CHI = EI_P // NW // KI
EO_P = 16384
EPWO = EO_P // NW
KO = 128
CHO = EPWO // KO
RSI = NPI // NW
KSI = 80
CSI = RSI // KSI
RSO = NPO // NW
KSO = 32
CSO = 1

_mesh = plsc.VectorSubcoreMesh(core_axis_name="c", subcore_axis_name="s")
_sc_params = pltpu.CompilerParams()
if "needs_layout_passes" in pltpu.CompilerParams.__dataclass_fields__:
    _sc_params = dataclasses.replace(_sc_params, needs_layout_passes=False)
f32 = jnp.float32
i32 = jnp.int32


def _hist_body(src_i_hbm, dst_i_hbm, src_o_hbm, dst_o_hbm, val_o_hbm,
               ids_i_hbm, val_ii_hbm, ids_o_hbm, val_io_hbm,
               do_i_hbm, di_i_hbm, ci_hbm, do_o_hbm, di_o_hbm, co_hbm,
               ebuf_s, ebuf_d, obuf_s, obuf_d, obuf_v, ibuf, ivbuf,
               obuf2, ovbuf2,
               h_do_i, h_di_i, h_ci, h_do_o, h_di_o, h_co):
    w = lax.axis_index("c") * NS + lax.axis_index("s")
    ones = jnp.full((16,), 1.0, f32)
    zeros = jnp.zeros((16,), f32)

    @pl.loop(0, NPI // 16)
    def _(i):
        h_do_i[pl.ds(i * 16, 16)] = zeros
        h_di_i[pl.ds(i * 16, 16)] = zeros

    @pl.loop(0, NPO // 16)
    def _(i):
        h_ci[pl.ds(i * 16, 16)] = zeros
        h_do_o[pl.ds(i * 16, 16)] = zeros
        h_di_o[pl.ds(i * 16, 16)] = zeros

    @pl.loop(0, B // 16)
    def _(i):
        h_co[pl.ds(i * 16, 16)] = zeros

    pltpu.sync_copy(src_i_hbm.at[w], ebuf_s)
    pltpu.sync_copy(dst_i_hbm.at[w], ebuf_d)

    @pl.loop(0, EPWI // 16)
    def _(i):
        s16 = ebuf_s[pl.ds(i * 16, 16)]
        d16 = ebuf_d[pl.ds(i * 16, 16)]
        plsc.addupdate_scatter(h_do_i, [s16], ones)
        plsc.addupdate_scatter(h_di_i, [d16], ones)

    pltpu.sync_copy(src_o_hbm.at[w], obuf_s)
    pltpu.sync_copy(dst_o_hbm.at[w], obuf_d)
    pltpu.sync_copy(val_o_hbm.at[w], obuf_v)

    @pl.loop(0, EPWO // 16)
    def _(i):
        s16 = obuf_s[pl.ds(i * 16, 16)]
        d16 = obuf_d[pl.ds(i * 16, 16)]
        v16 = obuf_v[pl.ds(i * 16, 16)]
        plsc.addupdate_scatter(h_do_o, [s16], v16)
        plsc.addupdate_scatter(h_di_o, [d16], v16)

    pltpu.sync_copy(ids_i_hbm.at[w], ibuf)
    pltpu.sync_copy(val_ii_hbm.at[w], ivbuf)

    @pl.loop(0, RSI // 16)
    def _(i):
        g16 = ibuf[pl.ds(i * 16, 16)]
        v16 = ivbuf[pl.ds(i * 16, 16)]
        plsc.addupdate_scatter(h_ci, [g16], v16)

    pltpu.sync_copy(ids_o_hbm.at[w], obuf2)
    pltpu.sync_copy(val_io_hbm.at[w], ovbuf2)

    @pl.loop(0, RSO // 16)
    def _(i):
        g16 = obuf2[pl.ds(i * 16, 16)]
        v16 = ovbuf2[pl.ds(i * 16, 16)]
        plsc.addupdate_scatter(h_co, [g16], v16)

    pltpu.sync_copy(h_do_i, do_i_hbm.at[w])
    pltpu.sync_copy(h_di_i, di_i_hbm.at[w])
    pltpu.sync_copy(h_ci, ci_hbm.at[w])
    pltpu.sync_copy(h_do_o, do_o_hbm.at[w])
    pltpu.sync_copy(h_di_o, di_o_hbm.at[w])
    pltpu.sync_copy(h_co, co_hbm.at[w])


_hist_kernel = pl.kernel(
    _hist_body,
    out_type=(
        jax.ShapeDtypeStruct((NW, NPI), f32),
        jax.ShapeDtypeStruct((NW, NPI), f32),
        jax.ShapeDtypeStruct((NW, NPO), f32),
        jax.ShapeDtypeStruct((NW, NPO), f32),
        jax.ShapeDtypeStruct((NW, NPO), f32),
        jax.ShapeDtypeStruct((NW, B), f32),
    ),
    mesh=_mesh,
    compiler_params=_sc_params,
    scratch_types=[
        pltpu.VMEM((EPWI,), i32), pltpu.VMEM((EPWI,), i32),
        pltpu.VMEM((EPWO,), i32), pltpu.VMEM((EPWO,), i32),
        pltpu.VMEM((EPWO,), f32),
        pltpu.VMEM((RSI,), i32), pltpu.VMEM((RSI,), f32),
        pltpu.VMEM((RSO,), i32), pltpu.VMEM((RSO,), f32),
        pltpu.VMEM((NPI,), f32), pltpu.VMEM((NPI,), f32),
        pltpu.VMEM((NPO,), f32), pltpu.VMEM((NPO,), f32),
        pltpu.VMEM((NPO,), f32), pltpu.VMEM((B,), f32),
    ],
)


def _make_mp(np_, ch, k):
    rz = np_ // NS

    assert ch % 4 == 0

    def body(h_hbm, pk_hbm, zeros_hbm, out_hbm,
             pk0, pk1, pk2, pk3, rows0, rows1, acc_sh,
             sp0, sp1, sp2, sp3, sg0, sg1):
        c_id = lax.axis_index("c")
        s_id = lax.axis_index("s")
        w = c_id * NS + s_id
        pks = (pk0, pk1, pk2, pk3)
        rows = (rows0, rows1)
        sps = (sp0, sp1, sp2, sp3)
        sgs = (sg0, sg1)
        pltpu.sync_copy(zeros_hbm.at[pl.ds(s_id * rz, rz)],
                        acc_sh.at[pl.ds(s_id * rz, rz)])
        plsc.subcore_barrier()

        for q in range(4):
            pltpu.async_copy(pk_hbm.at[w].at[q], pks[q], sps[q])
        pltpu.make_async_copy(pk_hbm.at[w].at[0], pks[0], sps[0]).wait()
        pltpu.async_copy(h_hbm.at[pks[0].at[0]], rows[0], sgs[0])

        def do_chunk(c, q):
            b = q % 2
            nq, nb = (q + 1) % 4, (q + 1) % 2
            pltpu.make_async_copy(h_hbm.at[pks[q].at[0]], rows[b],
                                  sgs[b]).wait()
            pltpu.make_async_copy(pk_hbm.at[w].at[0], pks[nq],
                                  sps[nq]).wait()
            pltpu.async_copy(h_hbm.at[pks[nq].at[0]], rows[nb], sgs[nb])

            @pl.loop(0, k // 16)
            def _(ii):
                wv = plsc.bitcast(pks[q][2, pl.ds(ii * 16, 16)], f32)
                for r in range(16):
                    wgt = wv[r]
                    for j in range(D // 16):
                        sl = (ii * 16 + r, pl.ds(j * 16, 16))
                        rows[b][sl] = rows[b][sl] * wgt

            pltpu.sync_copy(rows[b], acc_sh.at[pks[q].at[1]], add=True)
            pltpu.async_copy(pk_hbm.at[w].at[c + 4], pks[q], sps[q])

        @pl.loop(0, ch // 4)
        def _(i):
            for q in range(4):
                do_chunk(4 * i + q, q)

        pltpu.make_async_copy(h_hbm.at[pks[0].at[0]], rows[0], sgs[0]).wait()
        for q in range(1, 4):
            pltpu.make_async_copy(pk_hbm.at[w].at[0], pks[q], sps[q]).wait()

        plsc.subcore_barrier()
        pltpu.sync_copy(acc_sh.at[pl.ds(s_id * rz, rz)],
                        out_hbm.at[c_id].at[pl.ds(s_id * rz, rz)])

    return pl.kernel(
        body,
        out_type=jax.ShapeDtypeStruct((NC, np_, D), f32),
        mesh=_mesh,
        compiler_params=_sc_params,
        scratch_types=[
            pltpu.VMEM((3, k), i32), pltpu.VMEM((3, k), i32),
            pltpu.VMEM((3, k), i32), pltpu.VMEM((3, k), i32),
            pltpu.VMEM((k, D), f32), pltpu.VMEM((k, D), f32),
            pltpu.VMEM_SHARED((np_, D), f32),
            pltpu.SemaphoreType.DMA, pltpu.SemaphoreType.DMA,
            pltpu.SemaphoreType.DMA, pltpu.SemaphoreType.DMA,
            pltpu.SemaphoreType.DMA, pltpu.SemaphoreType.DMA,
        ],
    )


_mp_inner = _make_mp(NPI, CHI, KI)
_mp_outer = _make_mp(NPO, CHO, KO)


def _make_segsum(np_rows, m, cs, k):
    rpw = np_rows // NW
    rz = m // NS

    def body(h_hbm, ids_hbm, zeros_hbm, out_hbm, ids_v, rows_v, acc_sh):
        c_id = lax.axis_index("c")
        s_id = lax.axis_index("s")
        w = c_id * NS + s_id
        pltpu.sync_copy(zeros_hbm.at[pl.ds(s_id * rz, rz)],
                        acc_sh.at[pl.ds(s_id * rz, rz)])
        pltpu.sync_copy(ids_hbm.at[w], ids_v)
        plsc.subcore_barrier()

        @pl.loop(0, cs)
        def _(c):
            pltpu.sync_copy(h_hbm.at[pl.ds(w * rpw + c * k, k)], rows_v)
            pltpu.sync_copy(rows_v, acc_sh.at[ids_v.at[c]], add=True)

        plsc.subcore_barrier()
        pltpu.sync_copy(acc_sh.at[pl.ds(s_id * rz, rz)],
                        out_hbm.at[c_id].at[pl.ds(s_id * rz, rz)])

    return pl.kernel(
        body,
        out_type=jax.ShapeDtypeStruct((NC, m, D), f32),
        mesh=_mesh,
        compiler_params=_sc_params,
        scratch_types=[
            pltpu.VMEM((cs, k), i32), pltpu.VMEM((k, D), f32),
            pltpu.VMEM_SHARED((m, D), f32),
        ],
    )


_segsum_inner = _make_segsum(NPI, NPO, CSI, KSI)
_segsum_outer = _make_segsum(NPO, B, CSO, KSO)


def _t0_body(do_i, di_i, ci, do_o, di_o, co, x, w1,
             h1_o, nsi_o, ndi_o, ici_o, nso_o, ndo_o, ico_o):
    deg_out_i = jnp.sum(do_i[...], axis=0)
    deg_in_i = jnp.sum(di_i[...], axis=0)
    cnt_i = jnp.sum(ci[...], axis=0)
    deg_out_o = jnp.sum(do_o[...], axis=0)
    deg_in_o = jnp.sum(di_o[...], axis=0)
    cnt_o = jnp.sum(co[...], axis=0)
    nsi = lax.rsqrt(jnp.maximum(deg_out_i, 1.0))
    ndi = lax.rsqrt(jnp.maximum(deg_in_i, 1.0))
    nso = lax.rsqrt(jnp.maximum(deg_out_o, 1.0))
    ndo = lax.rsqrt(jnp.maximum(deg_in_o, 1.0))
    nsi_o[...] = nsi
    ndi_o[...] = ndi
    ici_o[...] = 1.0 / jnp.maximum(cnt_i, 1.0)
    nso_o[...] = nso
    ndo_o[...] = ndo
    ico_o[...] = 1.0 / jnp.maximum(cnt_o, 1.0)
    h1_o[...] = jnp.dot(x[...] * nsi[:, None], w1[...],
                        preferred_element_type=f32)


def _tc_t0(do_i, di_i, ci, do_o, di_o, co, x, w1):
    return pl.pallas_call(
        _t0_body,
        out_shape=(
            jax.ShapeDtypeStruct((NPI, D), f32),
            jax.ShapeDtypeStruct((NPI,), f32),
            jax.ShapeDtypeStruct((NPI,), f32),
            jax.ShapeDtypeStruct((NPO,), f32),
            jax.ShapeDtypeStruct((NPO,), f32),
            jax.ShapeDtypeStruct((NPO,), f32),
            jax.ShapeDtypeStruct((B,), f32),
        ),
    )(do_i, di_i, ci, do_o, di_o, co, x, w1)


def _tmid_body(p, nd, ns, b, w, o):
    agg = p[0] + p[1]
    x = jax.nn.relu(agg * nd[...][:, None] + b[...][None, :]) * ns[...][:, None]
    o[...] = jnp.dot(x, w[...], preferred_element_type=f32)


def _tc_mid(p, nd, ns, b, w):
    n = p.shape[1]
    return pl.pallas_call(
        _tmid_body,
        out_shape=jax.ShapeDtypeStruct((n, w.shape[1]), f32),
    )(p, nd, ns, b, w)


def _tmask_body(p, nd, b, nreal_ref, o):
    agg = p[0] + p[1]
    h = jax.nn.relu(agg * nd[...][:, None] + b[...][None, :])
    rows = lax.broadcasted_iota(i32, h.shape, 0)
    o[...] = jnp.where(rows < nreal_ref[0], h, 0.0)


def _tc_mask(p, nd, b, nreal):
    n = p.shape[1]
    return pl.pallas_call(
        _tmask_body,
        out_shape=jax.ShapeDtypeStruct((n, D), f32),
    )(p, nd, b, jnp.full((1,), nreal, i32))


def _t3_body(q, ici, feat, nso, w3, o):
    red = (q[0] + q[1]) * ici[...][:, None]
    merged = jnp.concatenate([feat[...], red], axis=1)
    o[...] = jnp.dot(merged * nso[...][:, None], w3[...],
                     preferred_element_type=f32)


def _tc_t3(q, ici, feat, nso, w3):
    return pl.pallas_call(
        _t3_body,
        out_shape=jax.ShapeDtypeStruct((NPO, D), f32),
    )(q, ici, feat, nso, w3)


def _t8_body(s, ico, wc, bc, o):
    hg = (s[0] + s[1]) * ico[...][:, None]
    o[...] = jnp.dot(hg, wc[...], preferred_element_type=f32) + bc[...]


def _tc_t8(s, ico, wc, bc):
    return pl.pallas_call(
        _t8_body,
        out_shape=jax.ShapeDtypeStruct((B, NUM_CLASSES), f32),
    )(s, ico, wc, bc[None, :])


def kernel(in_layer_feat, out_layer_feat, edge_index_in, edge_index_out,
           node_graph_ids_in, graph_ids_out, inner_edge_weight,
           outer_edge_weight, W1, b1, W2, b2, W3, b3, W4, b4, W5, b5,
           W6, b6, Wc, bc):
    src_i = edge_index_in[0].astype(i32)
    dst_i = edge_index_in[1].astype(i32)
    src_o = edge_index_out[0].astype(i32)
    dst_o = edge_index_out[1].astype(i32)

    x_p = jnp.pad(in_layer_feat, ((0, NPI - N_IN), (0, 0)))
    feat_o_p = jnp.pad(out_layer_feat, ((0, NPO - N_OUT), (0, 0)))
    zeros_big = jnp.zeros((NPI, D), f32)

    pad_ei = EI_P - E_IN
    spread_i = N_IN + (jnp.arange(pad_ei, dtype=i32) % (NPI - N_IN))
    src_ip = jnp.concatenate([src_i, spread_i])
    dst_ip = jnp.concatenate([dst_i, spread_i])
    ew_ip = jnp.pad(inner_edge_weight.astype(f32), (0, pad_ei))
    ew_i_bits = lax.bitcast_convert_type(ew_ip, i32)
    pk_i = jnp.stack([src_ip.reshape(NW, CHI, KI),
                      dst_ip.reshape(NW, CHI, KI),
                      ew_i_bits.reshape(NW, CHI, KI)], axis=2)
    dummy_i = jnp.broadcast_to(
        (jnp.arange(KI, dtype=i32) * 79) % NPI, (NW, 4, 3, KI))
    pk_i = jnp.concatenate([pk_i, dummy_i], axis=1)
    srcs_i2 = src_i.reshape(NW, EPWI)
    dsts_i2 = dst_i.reshape(NW, EPWI)

    pad_e = EO_P - E_OUT
    spread_o = N_OUT + (jnp.arange(pad_e, dtype=i32) % (NPO - N_OUT))
    src_o_p = jnp.concatenate([src_o, spread_o])
    dst_o_p = jnp.concatenate([dst_o, spread_o])
    ew_o_p = jnp.pad(outer_edge_weight.astype(f32), (0, pad_e))
    val_o_p = jnp.pad(jnp.ones((E_OUT,), f32), (0, pad_e))
    ew_o_bits = lax.bitcast_convert_type(ew_o_p, i32)
    pk_o = jnp.stack([src_o_p.reshape(NW, CHO, KO),
                      dst_o_p.reshape(NW, CHO, KO),
                      ew_o_bits.reshape(NW, CHO, KO)], axis=2)
    dummy_o = jnp.broadcast_to(
        (jnp.arange(KO, dtype=i32) * 31) % NPO, (NW, 4, 3, KO))
    pk_o = jnp.concatenate([pk_o, dummy_o], axis=1)
    srcs_o2 = src_o_p.reshape(NW, EPWO)
    dsts_o2 = dst_o_p.reshape(NW, EPWO)
    vals_o2 = val_o_p.reshape(NW, EPWO)

    ids_i_p = jnp.concatenate([
        node_graph_ids_in.astype(i32),
        jnp.arange(NPI - N_IN, dtype=i32) % NPO])
    val_ii = jnp.pad(jnp.ones((N_IN,), f32), (0, NPI - N_IN))
    ids_o_p = jnp.concatenate([
        graph_ids_out.astype(i32),
        jnp.arange(NPO - N_OUT, dtype=i32) % B])
    val_io = jnp.pad(jnp.ones((N_OUT,), f32), (0, NPO - N_OUT))
    ids_i2 = ids_i_p.reshape(NW, RSI)
    val_ii2 = val_ii.reshape(NW, RSI)
    ids_o2 = ids_o_p.reshape(NW, RSO)
    val_io2 = val_io.reshape(NW, RSO)
    ids_i3 = ids_i_p.reshape(NW, CSI, KSI)
    ids_o3 = ids_o_p.reshape(NW, CSO, KSO)

    do_i, di_i, ci, do_o, di_o, co = _hist_kernel(
        srcs_i2, dsts_i2, srcs_o2, dsts_o2, vals_o2,
        ids_i2, val_ii2, ids_o2, val_io2)

    h1, nsi, ndi, ici, nso, ndo, ico = _tc_t0(
        do_i, di_i, ci, do_o, di_o, co, x_p, W1)

    p1 = _mp_inner(h1, pk_i, zeros_big)
    h2 = _tc_mid(p1, ndi, nsi, b1, W2)
    p2 = _mp_inner(h2, pk_i, zeros_big)
    hin = _tc_mask(p2, ndi, b2, N_IN)

    q = _segsum_inner(hin, ids_i3, zeros_big)
    h3 = _tc_t3(q, ici, feat_o_p, nso, W3)

    p3 = _mp_outer(h3, pk_o, zeros_big)
    h4 = _tc_mid(p3, ndo, nso, b3, W4)
    p4 = _mp_outer(h4, pk_o, zeros_big)
    h5 = _tc_mid(p4, ndo, nso, b4, W5)
    p5 = _mp_outer(h5, pk_o, zeros_big)
    h6 = _tc_mid(p5, ndo, nso, b5, W6)
    p6 = _mp_outer(h6, pk_o, zeros_big)
    hout = _tc_mask(p6, ndo, b6, N_OUT)

    s = _segsum_outer(hout, ids_o3, zeros_big)
    return _tc_t8(s, ico, Wc, bc)
